# baseline (speedup 1.0000x reference)
import functools

import jax
import jax.numpy as jnp
from jax import lax
from jax.experimental import pallas as pl
from jax.experimental.pallas import tpu as pltpu
from jax.experimental.pallas import tpu_sc as plsc

_COLS = 128
_ROWS = 65536
_BLOCK_ROWS = 16384
_PATCH = 1024


def _sc_patch_body(idx_hbm, val_hbm, patch_hbm, pbuf, idxv, valv):
    c = lax.axis_index("c")
    s = lax.axis_index("s")
    wid = s * 2 + c

    @pl.when(wid == 0)
    def _():
        zero = jnp.zeros((16,), jnp.float32)
        for k in range(_PATCH // 16):
            pbuf[pl.ds(k * 16, 16)] = zero
        pltpu.sync_copy(idx_hbm, idxv.at[pl.ds(0, 4)])
        pltpu.sync_copy(val_hbm, valv.at[pl.ds(0, 4)])
        mask = lax.iota(jnp.int32, 16) < 4
        iv = jnp.where(mask, idxv[...], 0)
        plsc.addupdate_scatter(pbuf, [iv], valv[...], mask=mask)
        pltpu.sync_copy(pbuf, patch_hbm)


_sc_patch = functools.partial(
    pl.kernel,
    out_type=jax.ShapeDtypeStruct((_PATCH,), jnp.float32),
    mesh=plsc.VectorSubcoreMesh(core_axis_name="c", subcore_axis_name="s",
                                num_cores=2, num_subcores=16),
    scratch_types=[
        pltpu.VMEM((_PATCH,), jnp.float32),
        pltpu.VMEM((16,), jnp.int32),
        pltpu.VMEM((16,), jnp.float32),
    ],
    compiler_params=pltpu.CompilerParams(needs_layout_passes=False),
)


def _tc_body(patch_ref, in_ref, out_ref):
    out_ref[...] = in_ref[...]

    @pl.when(pl.program_id(0) == 0)
    def _():
        out_ref[0:8, :] += patch_ref[...]


def kernel(a, indices, values):
    n = a.shape[0]
    a2 = a.reshape(_ROWS, _COLS)
    idx = indices.astype(jnp.int32)
    vals = values.reshape(-1)

    patch = _sc_patch(_sc_patch_body)(idx, vals).reshape(8, _COLS)

    out = pl.pallas_call(
        _tc_body,
        grid=(_ROWS // _BLOCK_ROWS,),
        in_specs=[
            pl.BlockSpec((8, _COLS), lambda i: (0, 0)),
            pl.BlockSpec((_BLOCK_ROWS, _COLS), lambda i: (i, 0)),
        ],
        out_specs=pl.BlockSpec((_BLOCK_ROWS, _COLS), lambda i: (i, 0)),
        out_shape=jax.ShapeDtypeStruct((_ROWS, _COLS), jnp.float32),
        compiler_params=pltpu.CompilerParams(
            dimension_semantics=("parallel",),
        ),
    )(patch, a2)
    return out.reshape(n, 1)


# speedup vs baseline: 1.0981x; 1.0981x over previous
import functools

import jax
import jax.numpy as jnp
from jax import lax
from jax.experimental import pallas as pl
from jax.experimental.pallas import tpu as pltpu
from jax.experimental.pallas import tpu_sc as plsc

_COLS = 128
_ROWS = 65536
_BLOCK_ROWS = 16384
_PATCH = 1024


def _sc_patch_body(idx_hbm, val_hbm, patch_hbm, pbuf, idxv, valv):
    c = lax.axis_index("c")
    s = lax.axis_index("s")
    wid = s * 2 + c

    @pl.when(wid == 0)
    def _():
        zero = jnp.zeros((16,), jnp.float32)
        for k in range(_PATCH // 16):
            pbuf[pl.ds(k * 16, 16)] = zero
        pltpu.sync_copy(idx_hbm, idxv.at[pl.ds(0, 4)])
        pltpu.sync_copy(val_hbm, valv.at[pl.ds(0, 4)])
        mask = lax.iota(jnp.int32, 16) < 4
        iv = jnp.where(mask, idxv[...], 0)
        plsc.addupdate_scatter(pbuf, [iv], valv[...], mask=mask)
        pltpu.sync_copy(pbuf, patch_hbm)


_sc_patch = functools.partial(
    pl.kernel,
    out_type=jax.ShapeDtypeStruct((_PATCH,), jnp.float32),
    mesh=plsc.VectorSubcoreMesh(core_axis_name="c", subcore_axis_name="s",
                                num_cores=2, num_subcores=16),
    scratch_types=[
        pltpu.VMEM((_PATCH,), jnp.float32),
        pltpu.VMEM((16,), jnp.int32),
        pltpu.VMEM((16,), jnp.float32),
    ],
    compiler_params=pltpu.CompilerParams(needs_layout_passes=False),
)


def _copy_body(in_ref, out_ref):
    out_ref[...] = in_ref[...]


def _merge_body(in_ref, patch_ref, out_ref):
    out_ref[...] = in_ref[...] + patch_ref[...]


def kernel(a, indices, values):
    n = a.shape[0]
    a2 = a.reshape(_ROWS, _COLS)
    idx = indices.astype(jnp.int32)
    vals = values.reshape(-1)

    patch = _sc_patch(_sc_patch_body)(idx, vals).reshape(8, _COLS)

    copied = pl.pallas_call(
        _copy_body,
        grid=(_ROWS // _BLOCK_ROWS,),
        in_specs=[pl.BlockSpec((_BLOCK_ROWS, _COLS), lambda i: (i, 0))],
        out_specs=pl.BlockSpec((_BLOCK_ROWS, _COLS), lambda i: (i, 0)),
        out_shape=jax.ShapeDtypeStruct((_ROWS, _COLS), jnp.float32),
        compiler_params=pltpu.CompilerParams(
            dimension_semantics=("parallel",),
        ),
    )(a2)

    out = pl.pallas_call(
        _merge_body,
        grid=(1,),
        in_specs=[
            pl.BlockSpec((8, _COLS), lambda i: (0, 0)),
            pl.BlockSpec((8, _COLS), lambda i: (0, 0)),
        ],
        out_specs=pl.BlockSpec((8, _COLS), lambda i: (0, 0)),
        out_shape=jax.ShapeDtypeStruct((_ROWS, _COLS), jnp.float32),
        input_output_aliases={0: 0},
    )(copied, patch)
    return out.reshape(n, 1)


# speedup vs baseline: 1.8585x; 1.6925x over previous
import jax
import jax.numpy as jnp
from jax.experimental import pallas as pl
from jax.experimental.pallas import tpu as pltpu

_COLS = 128
_BLOCK_ROWS = 16384


def _body(idx_ref, val_ref, in_ref, out_ref):
    out_ref[...] = in_ref[...]

    @pl.when(pl.program_id(0) == 0)
    def _():
        row_i = jax.lax.broadcasted_iota(jnp.int32, (8, _COLS), 0)
        col_i = jax.lax.broadcasted_iota(jnp.int32, (8, _COLS), 1)
        flat = row_i * _COLS + col_i
        acc = jnp.zeros((8, _COLS), jnp.float32)
        for i in range(4):
            acc += jnp.where(flat == idx_ref[i], val_ref[i, 0], 0.0)
        out_ref[0:8, :] += acc


def kernel(a, indices, values):
    n = a.shape[0]
    rows = n // _COLS
    a2 = a.reshape(rows, _COLS)
    idx = indices.astype(jnp.int32)

    out = pl.pallas_call(
        _body,
        grid=(rows // _BLOCK_ROWS,),
        in_specs=[
            pl.BlockSpec(memory_space=pltpu.SMEM),
            pl.BlockSpec(memory_space=pltpu.SMEM),
            pl.BlockSpec((_BLOCK_ROWS, _COLS), lambda i: (i, 0)),
        ],
        out_specs=pl.BlockSpec((_BLOCK_ROWS, _COLS), lambda i: (i, 0)),
        out_shape=jax.ShapeDtypeStruct((rows, _COLS), jnp.float32),
        compiler_params=pltpu.CompilerParams(
            dimension_semantics=("parallel",),
        ),
    )(idx, values, a2)
    return out.reshape(n, 1)
